# Initial kernel scaffold; baseline (speedup 1.0000x reference)
import jax
import jax.numpy as jnp
from jax.experimental import pallas as pl


def kernel(h, edge_index):
    raise NotImplementedError("write your pallas kernel here")


# speedup vs baseline: 132.3939x; 132.3939x over previous
import functools

import jax
import jax.numpy as jnp
from jax import lax
from jax.experimental import pallas as pl
from jax.experimental.pallas import tpu as pltpu
from jax.experimental.pallas import tpu_sc as plsc

N = 100000
NC = 2
NS = 16
LANES = 16

NPAD = 100352
NSL = NPAD // NS
VLOOP = NSL // LANES

E = 6400000
WIN = 128
BLK = 16
EPAD = 6422528
EROWS = EPAD // WIN
TILE_WROWS = EROWS // (NC * NS)
NBLKS = TILE_WROWS // BLK

_mesh = plsc.VectorSubcoreMesh(
    core_axis_name="c", subcore_axis_name="s", num_cores=NC, num_subcores=NS)

_f32 = jnp.float32
_i32 = jnp.int32


def _zero_loop(ref):
  def body(j, _):
    ref[pl.ds(j * LANES, LANES)] = jnp.zeros((LANES,), _f32)
    return 0
  lax.fori_loop(0, VLOOP, body, 0)


def _edge_phase(cid, sid, roww, colw, y_s, acc_s, rbuf, cbuf, vbuf):
  wrbase = (cid * NS + sid) * TILE_WROWS

  def blk_body(b, _):
    wr = wrbase + b * BLK
    pltpu.sync_copy(roww.at[pl.ds(wr, BLK), :], rbuf)
    pltpu.sync_copy(colw.at[pl.ds(wr, BLK), :], cbuf)
    for w in range(BLK):
      pltpu.sync_copy(y_s.at[rbuf.at[w]], vbuf.at[w])
    for w in range(BLK):
      pltpu.sync_copy(vbuf.at[w], acc_s.at[cbuf.at[w]], add=True)
    return 0

  lax.fori_loop(0, NBLKS, blk_body, 0)


def _hist_body(colw, dpart, deg_s, zbuf, cbuf, ones):
  cid = lax.axis_index("c")
  sid = lax.axis_index("s")
  base = sid * NSL

  _zero_loop(zbuf)
  for w in range(WIN // LANES):
    ones[pl.ds(w * LANES, LANES)] = jnp.ones((LANES,), _f32)
  pltpu.sync_copy(zbuf, deg_s.at[pl.ds(base, NSL)])
  plsc.subcore_barrier()

  wrbase = (cid * NS + sid) * TILE_WROWS

  def blk_body(b, _):
    wr = wrbase + b * BLK
    pltpu.sync_copy(colw.at[pl.ds(wr, BLK), :], cbuf)
    for w in range(BLK):
      pltpu.sync_copy(ones, deg_s.at[cbuf.at[w]], add=True)
    return 0

  lax.fori_loop(0, NBLKS, blk_body, 0)
  plsc.subcore_barrier()
  pltpu.sync_copy(deg_s.at[pl.ds(base, NSL)], dpart.at[cid, pl.ds(base, NSL)])


def _layer_body(first, *refs):
  if first:
    (roww, colw, y0, zpart,
     y_s, acc_s, ab, bb, cb, db, ob, rbuf, cbuf, vbuf, ps) = refs
  else:
    (roww, colw, pa, yp, dv, y_o, psum, zpart,
     y_s, acc_s, ab, bb, cb, db, ob, rbuf, cbuf, vbuf, ps) = refs
  cid = lax.axis_index("c")
  sid = lax.axis_index("s")
  base = sid * NSL
  sl = pl.ds(base, NSL)

  if first:
    pltpu.sync_copy(y0.at[sl], y_s.at[sl])
  else:
    pltpu.sync_copy(pa.at[0, sl], ab)
    pltpu.sync_copy(pa.at[1, sl], bb)
    pltpu.sync_copy(yp.at[sl], cb)
    pltpu.sync_copy(dv.at[sl], db)
    ps[...] = jnp.zeros((LANES,), _f32)

    def ew_body(j, _):
      v = pl.ds(j * LANES, LANES)
      dvv = db[v]
      x = dvv * (ab[v] + bb[v] + cb[v])
      ps[...] = ps[...] + x * x
      ob[v] = dvv * x
      return 0

    lax.fori_loop(0, VLOOP, ew_body, 0)

    pltpu.sync_copy(ob, y_s.at[sl])

    @pl.when(cid == 0)
    def _():
      pltpu.sync_copy(ob, y_o.at[sl])
      pltpu.sync_copy(ps, psum.at[sid])

  zb = bb if first else ab
  _zero_loop(zb)
  pltpu.sync_copy(zb, acc_s.at[sl])
  plsc.subcore_barrier()

  _edge_phase(cid, sid, roww, colw, y_s, acc_s, rbuf, cbuf, vbuf)

  plsc.subcore_barrier()
  pltpu.sync_copy(acc_s.at[sl], zpart.at[cid, sl])


def _final_body(pa, yp, dv, psum, ab, bb, cb, db, ps):
  cid = lax.axis_index("c")
  sid = lax.axis_index("s")
  sl = pl.ds(sid * NSL, NSL)
  pltpu.sync_copy(pa.at[0, sl], ab)
  pltpu.sync_copy(pa.at[1, sl], bb)
  pltpu.sync_copy(yp.at[sl], cb)
  pltpu.sync_copy(dv.at[sl], db)
  ps[...] = jnp.zeros((LANES,), _f32)

  def ew_body(j, _):
    v = pl.ds(j * LANES, LANES)
    x = db[v] * (ab[v] + bb[v] + cb[v])
    ps[...] = ps[...] + x * x
    return 0

  lax.fori_loop(0, VLOOP, ew_body, 0)

  @pl.when(cid == 0)
  def _():
    pltpu.sync_copy(ps, psum.at[sid])


_node_f32 = jax.ShapeDtypeStruct((NPAD,), _f32)
_part_f32 = jax.ShapeDtypeStruct((NC, NPAD), _f32)
_psum_t = jax.ShapeDtypeStruct((NS, LANES), _f32)

_edge_scratch = [
    pltpu.MemorySpace.VMEM_SHARED((NPAD,), _f32),
    pltpu.MemorySpace.VMEM_SHARED((NPAD,), _f32),
    pltpu.VMEM((NSL,), _f32),
    pltpu.VMEM((NSL,), _f32),
    pltpu.VMEM((NSL,), _f32),
    pltpu.VMEM((NSL,), _f32),
    pltpu.VMEM((NSL,), _f32),
    pltpu.VMEM((BLK, WIN), _i32),
    pltpu.VMEM((BLK, WIN), _i32),
    pltpu.VMEM((BLK, WIN), _f32),
    pltpu.VMEM((LANES,), _f32),
]

_hist_kernel = pl.kernel(
    _hist_body,
    out_type=_part_f32,
    mesh=_mesh,
    scratch_types=[
        pltpu.MemorySpace.VMEM_SHARED((NPAD,), _f32),
        pltpu.VMEM((NSL,), _f32),
        pltpu.VMEM((BLK, WIN), _i32),
        pltpu.VMEM((WIN,), _f32),
    ],
)

_layer1_kernel = pl.kernel(
    functools.partial(_layer_body, True),
    out_type=_part_f32,
    mesh=_mesh,
    scratch_types=_edge_scratch,
)


def _dinv_tc_body(dp_ref, hp_ref, dinv_ref, y0_ref):
  d = dp_ref[0] + dp_ref[1] + 1.0
  dv = jax.lax.rsqrt(d)
  dinv_ref[...] = dv
  y0_ref[...] = dv * hp_ref[...]


_R2 = NPAD // 128
_dinv_tc = pl.pallas_call(
    _dinv_tc_body,
    out_shape=(
        jax.ShapeDtypeStruct((_R2, 128), _f32),
        jax.ShapeDtypeStruct((_R2, 128), _f32),
    ),
)

_layer_kernel = pl.kernel(
    functools.partial(_layer_body, False),
    out_type=(_node_f32, _psum_t, _part_f32),
    mesh=_mesh,
    scratch_types=_edge_scratch,
)

_final_kernel = pl.kernel(
    _final_body,
    out_type=_psum_t,
    mesh=_mesh,
    scratch_types=[
        pltpu.VMEM((NSL,), _f32),
        pltpu.VMEM((NSL,), _f32),
        pltpu.VMEM((NSL,), _f32),
        pltpu.VMEM((NSL,), _f32),
        pltpu.VMEM((LANES,), _f32),
    ],
)


@jax.jit
def kernel(h, edge_index):
  epad = EPAD - E
  fill = jnp.arange(epad, dtype=_i32)
  row = jnp.concatenate([edge_index[0], fill % N])
  col = jnp.concatenate([edge_index[1], N + fill % (NPAD - N)])
  roww = row.reshape(EROWS, WIN)
  colw = col.reshape(EROWS, WIN)
  hp = jnp.concatenate([h[:, 0], jnp.zeros((NPAD - N,), _f32)])

  dpart = _hist_kernel(colw)
  dinv2, y02 = _dinv_tc(dpart.reshape(NC, _R2, 128), hp.reshape(_R2, 128))
  dinv = dinv2.reshape(NPAD)
  y0 = y02.reshape(NPAD)
  z1 = _layer1_kernel(roww, colw, y0)
  y1, ps1, z2 = _layer_kernel(roww, colw, z1, y0, dinv)
  y2, ps2, z3 = _layer_kernel(roww, colw, z2, y1, dinv)
  ps3 = _final_kernel(z3, y2, dinv)
  return jnp.stack([jnp.sum(ps1), jnp.sum(ps2), jnp.sum(ps3)])


# speedup vs baseline: 386.3856x; 2.9185x over previous
import functools

import jax
import jax.numpy as jnp
from jax import lax
from jax.experimental import pallas as pl
from jax.experimental.pallas import tpu as pltpu
from jax.experimental.pallas import tpu_sc as plsc

N = 100000
NC = 2
NS = 16
LANES = 16

NPAD = 100352
NSL = NPAD // NS
VLOOP = NSL // LANES

E = 6400000
WIN = 128
BLK = 16
NBUF = 4
NBLKS = 100
EPAD = NC * NS * NBLKS * BLK * WIN
EROWS = EPAD // WIN
TILE_WROWS = EROWS // (NC * NS)

_mesh = plsc.VectorSubcoreMesh(
    core_axis_name="c", subcore_axis_name="s", num_cores=NC, num_subcores=NS)

_f32 = jnp.float32
_i32 = jnp.int32


def _zero_loop(ref):
  def body(j, _):
    ref[pl.ds(j * LANES, LANES)] = jnp.zeros((LANES,), _f32)
    return 0
  lax.fori_loop(0, VLOOP, body, 0)


def _edge_phase(cid, sid, roww, colw, y_s, acc_s, rbufs, cbufs, vbufs,
                semL, semS, semG, ones=None):
  hist = ones is not None
  wrbase = (cid * NS + sid) * TILE_WROWS

  def load(b, p):
    wr = wrbase + b * BLK
    if not hist:
      pltpu.async_copy(roww.at[pl.ds(wr, BLK), :], rbufs.at[p], semL[p])
    pltpu.async_copy(colw.at[pl.ds(wr, BLK), :], cbufs.at[p], semL[p])

  def wait_load(b, p):
    wr = wrbase + b * BLK
    if not hist:
      pltpu.make_async_copy(
          roww.at[pl.ds(wr, BLK), :], rbufs.at[p], semL[p]).wait()
    pltpu.make_async_copy(
        colw.at[pl.ds(wr, BLK), :], cbufs.at[p], semL[p]).wait()

  def gathers(p):
    def fire(w, _):
      pltpu.async_copy(y_s.at[rbufs.at[p, w]], vbufs.at[p, w], semG)
      return 0
    lax.fori_loop(0, BLK, fire, 0)

    def drain(w, _):
      pltpu.make_async_copy(
          y_s.at[rbufs.at[p, w]], vbufs.at[p, w], semG).wait()
      return 0
    lax.fori_loop(0, BLK, drain, 0)

  def scatters_fire(p):
    def fire(w, _):
      src = ones if hist else vbufs.at[p, w]
      pltpu.async_copy(src, acc_s.at[cbufs.at[p, w]], semS[p], add=True)
      return 0
    lax.fori_loop(0, BLK, fire, 0)

  def scatters_drain(p):
    def drain(w, _):
      src = ones if hist else vbufs.at[p, w]
      pltpu.make_async_copy(src, acc_s.at[cbufs.at[p, w]], semS[p]).wait()
      return 0
    lax.fori_loop(0, BLK, drain, 0)

  for b in range(NBUF):
    load(b, b)

  def g_body(g, _):
    for par in range(NBUF):
      c = NBUF * g + par
      p = (par + 1) % NBUF

      @pl.when(c >= NBUF - 1)
      def _():
        scatters_drain(p)

      nxt = c + 1

      @pl.when(jnp.logical_and(nxt >= NBUF, nxt < NBLKS))
      def _():
        load(nxt, p)

      wait_load(c, par)
      if not hist:
        gathers(par)
      scatters_fire(par)
    return 0

  lax.fori_loop(0, NBLKS // NBUF, g_body, 0)
  for par in range(1, NBUF):
    scatters_drain(par)


def _hist_body(colw, dpart, deg_s, zbuf, cbufs, ones, *sems):
  cid = lax.axis_index("c")
  sid = lax.axis_index("s")
  base = sid * NSL

  _zero_loop(zbuf)
  for w in range(WIN // LANES):
    ones[pl.ds(w * LANES, LANES)] = jnp.ones((LANES,), _f32)
  pltpu.sync_copy(zbuf, deg_s.at[pl.ds(base, NSL)])
  plsc.subcore_barrier()

  semL = sems[:NBUF]
  semS = sems[NBUF:2 * NBUF]
  _edge_phase(cid, sid, None, colw, None, deg_s, None, cbufs, None,
              semL, semS, None, ones=ones)

  plsc.subcore_barrier()
  pltpu.sync_copy(deg_s.at[pl.ds(base, NSL)], dpart.at[cid, pl.ds(base, NSL)])


def _layer_body(first, *refs):
  if first:
    (roww, colw, y0, zpart,
     y_s, acc_s, ab, bb, cb, db, ob, rbufs, cbufs, vbufs, ps, *sems) = refs
  else:
    (roww, colw, pa, yp, dv, y_o, psum, zpart,
     y_s, acc_s, ab, bb, cb, db, ob, rbufs, cbufs, vbufs, ps, *sems) = refs
  cid = lax.axis_index("c")
  sid = lax.axis_index("s")
  base = sid * NSL
  sl = pl.ds(base, NSL)

  if first:
    pltpu.sync_copy(y0.at[sl], y_s.at[sl])
  else:
    pltpu.sync_copy(pa.at[0, sl], ab)
    pltpu.sync_copy(pa.at[1, sl], bb)
    pltpu.sync_copy(yp.at[sl], cb)
    pltpu.sync_copy(dv.at[sl], db)
    ps[...] = jnp.zeros((LANES,), _f32)

    def ew_body(j, _):
      v = pl.ds(j * LANES, LANES)
      dvv = db[v]
      x = dvv * (ab[v] + bb[v] + cb[v])
      ps[...] = ps[...] + x * x
      ob[v] = dvv * x
      return 0

    lax.fori_loop(0, VLOOP, ew_body, 0)

    pltpu.sync_copy(ob, y_s.at[sl])

    @pl.when(cid == 0)
    def _():
      pltpu.sync_copy(ob, y_o.at[sl])
      pltpu.sync_copy(ps, psum.at[sid])

  zb = bb if first else ab
  _zero_loop(zb)
  pltpu.sync_copy(zb, acc_s.at[sl])
  plsc.subcore_barrier()

  _edge_phase(cid, sid, roww, colw, y_s, acc_s, rbufs, cbufs, vbufs,
              sems[:NBUF], sems[NBUF:2 * NBUF], sems[2 * NBUF])

  plsc.subcore_barrier()
  pltpu.sync_copy(acc_s.at[sl], zpart.at[cid, sl])


def _final_body(pa, yp, dv, psum, ab, bb, cb, db, ps):
  cid = lax.axis_index("c")
  sid = lax.axis_index("s")
  sl = pl.ds(sid * NSL, NSL)
  pltpu.sync_copy(pa.at[0, sl], ab)
  pltpu.sync_copy(pa.at[1, sl], bb)
  pltpu.sync_copy(yp.at[sl], cb)
  pltpu.sync_copy(dv.at[sl], db)
  ps[...] = jnp.zeros((LANES,), _f32)

  def ew_body(j, _):
    v = pl.ds(j * LANES, LANES)
    x = db[v] * (ab[v] + bb[v] + cb[v])
    ps[...] = ps[...] + x * x
    return 0

  lax.fori_loop(0, VLOOP, ew_body, 0)

  @pl.when(cid == 0)
  def _():
    pltpu.sync_copy(ps, psum.at[sid])


_node_f32 = jax.ShapeDtypeStruct((NPAD,), _f32)
_part_f32 = jax.ShapeDtypeStruct((NC, NPAD), _f32)
_psum_t = jax.ShapeDtypeStruct((NS, LANES), _f32)

_edge_scratch = [
    pltpu.MemorySpace.VMEM_SHARED((NPAD,), _f32),
    pltpu.MemorySpace.VMEM_SHARED((NPAD,), _f32),
    pltpu.VMEM((NSL,), _f32),
    pltpu.VMEM((NSL,), _f32),
    pltpu.VMEM((NSL,), _f32),
    pltpu.VMEM((NSL,), _f32),
    pltpu.VMEM((NSL,), _f32),
    pltpu.VMEM((NBUF, BLK, WIN), _i32),
    pltpu.VMEM((NBUF, BLK, WIN), _i32),
    pltpu.VMEM((NBUF, BLK, WIN), _f32),
    pltpu.VMEM((LANES,), _f32),
] + [pltpu.SemaphoreType.DMA] * (2 * NBUF + 1)

_hist_kernel = pl.kernel(
    _hist_body,
    out_type=_part_f32,
    mesh=_mesh,
    scratch_types=[
        pltpu.MemorySpace.VMEM_SHARED((NPAD,), _f32),
        pltpu.VMEM((NSL,), _f32),
        pltpu.VMEM((NBUF, BLK, WIN), _i32),
        pltpu.VMEM((WIN,), _f32),
    ] + [pltpu.SemaphoreType.DMA] * (2 * NBUF),
)

_layer1_kernel = pl.kernel(
    functools.partial(_layer_body, True),
    out_type=_part_f32,
    mesh=_mesh,
    scratch_types=_edge_scratch,
)


def _dinv_tc_body(dp_ref, hp_ref, dinv_ref, y0_ref):
  d = dp_ref[0] + dp_ref[1] + 1.0
  dv = jax.lax.rsqrt(d)
  dinv_ref[...] = dv
  y0_ref[...] = dv * hp_ref[...]


_R2 = NPAD // 128
_dinv_tc = pl.pallas_call(
    _dinv_tc_body,
    out_shape=(
        jax.ShapeDtypeStruct((_R2, 128), _f32),
        jax.ShapeDtypeStruct((_R2, 128), _f32),
    ),
)

_layer_kernel = pl.kernel(
    functools.partial(_layer_body, False),
    out_type=(_node_f32, _psum_t, _part_f32),
    mesh=_mesh,
    scratch_types=_edge_scratch,
)

_final_kernel = pl.kernel(
    _final_body,
    out_type=_psum_t,
    mesh=_mesh,
    scratch_types=[
        pltpu.VMEM((NSL,), _f32),
        pltpu.VMEM((NSL,), _f32),
        pltpu.VMEM((NSL,), _f32),
        pltpu.VMEM((NSL,), _f32),
        pltpu.VMEM((LANES,), _f32),
    ],
)


@jax.jit
def kernel(h, edge_index):
  epad = EPAD - E
  fill = jnp.arange(epad, dtype=_i32)
  row = jnp.concatenate([edge_index[0], fill % N])
  col = jnp.concatenate([edge_index[1], N + fill % (NPAD - N)])
  roww = row.reshape(EROWS, WIN)
  colw = col.reshape(EROWS, WIN)
  hp = jnp.concatenate([h[:, 0], jnp.zeros((NPAD - N,), _f32)])

  dpart = _hist_kernel(colw)
  dinv2, y02 = _dinv_tc(dpart.reshape(NC, _R2, 128), hp.reshape(_R2, 128))
  dinv = dinv2.reshape(NPAD)
  y0 = y02.reshape(NPAD)
  z1 = _layer1_kernel(roww, colw, y0)
  y1, ps1, z2 = _layer_kernel(roww, colw, z1, y0, dinv)
  y2, ps2, z3 = _layer_kernel(roww, colw, z2, y1, dinv)
  ps3 = _final_kernel(z3, y2, dinv)
  return jnp.stack([jnp.sum(ps1), jnp.sum(ps2), jnp.sum(ps3)])


# speedup vs baseline: 387.7176x; 1.0034x over previous
import functools

import jax
import jax.numpy as jnp
from jax import lax
from jax.experimental import pallas as pl
from jax.experimental.pallas import tpu as pltpu
from jax.experimental.pallas import tpu_sc as plsc

N = 100000
NC = 2
NS = 16
LANES = 16

NPAD = 100352
NSL = NPAD // NS
VLOOP = NSL // LANES

E = 6400000
WIN = 128
BLK = 16
NBUF = 4
NBLKS = 100
EBLK = BLK * WIN
EPAD = NC * NS * NBLKS * EBLK
EROWS = EPAD // EBLK

_mesh = plsc.VectorSubcoreMesh(
    core_axis_name="c", subcore_axis_name="s", num_cores=NC, num_subcores=NS)

_f32 = jnp.float32
_i32 = jnp.int32


def _zero_loop(ref):
  def body(j, _):
    ref[pl.ds(j * LANES, LANES)] = jnp.zeros((LANES,), _f32)
    return 0
  lax.fori_loop(0, VLOOP, body, 0)


def _edge_phase(cid, sid, roww, colw, y_s, acc_s, rbufs, cbufs, vbufs,
                semL, semS, semG, ones=None):
  hist = ones is not None
  wrbase = (cid * NS + sid) * NBLKS

  def load(b, p):
    wr = wrbase + b
    if not hist:
      pltpu.async_copy(roww.at[wr], rbufs[p], semL[p])
    pltpu.async_copy(colw.at[wr], cbufs[p], semL[p])

  def wait_load(b, p):
    wr = wrbase + b
    if not hist:
      pltpu.make_async_copy(roww.at[wr], rbufs[p], semL[p]).wait()
    pltpu.make_async_copy(colw.at[wr], cbufs[p], semL[p]).wait()

  def gathers(p):
    pltpu.async_copy(y_s.at[rbufs[p]], vbufs[p], semG)
    pltpu.make_async_copy(y_s.at[rbufs[p]], vbufs[p], semG).wait()

  def scatters_fire(p):
    src = ones if hist else vbufs[p]
    pltpu.async_copy(src, acc_s.at[cbufs[p]], semS[p], add=True)

  def scatters_drain(p):
    src = ones if hist else vbufs[p]
    pltpu.make_async_copy(src, acc_s.at[cbufs[p]], semS[p]).wait()

  for b in range(NBUF):
    load(b, b)

  def g_body(g, _):
    for par in range(NBUF):
      c = NBUF * g + par
      p = (par + 1) % NBUF

      @pl.when(c >= NBUF - 1)
      def _():
        scatters_drain(p)

      nxt = c + 1

      @pl.when(jnp.logical_and(nxt >= NBUF, nxt < NBLKS))
      def _():
        load(nxt, p)

      wait_load(c, par)
      if not hist:
        gathers(par)
      scatters_fire(par)
    return 0

  lax.fori_loop(0, NBLKS // NBUF, g_body, 0)
  for par in range(1, NBUF):
    scatters_drain(par)


def _hist_body(colw, dpart, deg_s, zbuf, *rest):
  cbufs = rest[:NBUF]
  ones = rest[NBUF]
  sems = rest[NBUF + 1:]
  cid = lax.axis_index("c")
  sid = lax.axis_index("s")
  base = sid * NSL

  _zero_loop(zbuf)

  def ones_body(w, _):
    ones[pl.ds(w * LANES, LANES)] = jnp.ones((LANES,), _f32)
    return 0

  lax.fori_loop(0, EBLK // LANES, ones_body, 0)
  pltpu.sync_copy(zbuf, deg_s.at[pl.ds(base, NSL)])
  plsc.subcore_barrier()

  semL = sems[:NBUF]
  semS = sems[NBUF:2 * NBUF]
  _edge_phase(cid, sid, None, colw, None, deg_s, None, cbufs, None,
              semL, semS, None, ones=ones)

  plsc.subcore_barrier()
  pltpu.sync_copy(deg_s.at[pl.ds(base, NSL)], dpart.at[cid, pl.ds(base, NSL)])


def _layer_body(first, *refs):
  if first:
    (roww, colw, y0, zpart,
     y_s, acc_s, ab, bb, cb, db, ob, *rest) = refs
  else:
    (roww, colw, pa, yp, dv, y_o, psum, zpart,
     y_s, acc_s, ab, bb, cb, db, ob, *rest) = refs
  rbufs = rest[:NBUF]
  cbufs = rest[NBUF:2 * NBUF]
  vbufs = rest[2 * NBUF:3 * NBUF]
  ps = rest[3 * NBUF]
  sems = rest[3 * NBUF + 1:]
  cid = lax.axis_index("c")
  sid = lax.axis_index("s")
  base = sid * NSL
  sl = pl.ds(base, NSL)

  if first:
    pltpu.sync_copy(y0.at[sl], y_s.at[sl])
  else:
    pltpu.sync_copy(pa.at[0, sl], ab)
    pltpu.sync_copy(pa.at[1, sl], bb)
    pltpu.sync_copy(yp.at[sl], cb)
    pltpu.sync_copy(dv.at[sl], db)
    ps[...] = jnp.zeros((LANES,), _f32)

    def ew_body(j, _):
      v = pl.ds(j * LANES, LANES)
      dvv = db[v]
      x = dvv * (ab[v] + bb[v] + cb[v])
      ps[...] = ps[...] + x * x
      ob[v] = dvv * x
      return 0

    lax.fori_loop(0, VLOOP, ew_body, 0)

    pltpu.sync_copy(ob, y_s.at[sl])

    @pl.when(cid == 0)
    def _():
      pltpu.sync_copy(ob, y_o.at[sl])
      pltpu.sync_copy(ps, psum.at[sid])

  zb = bb if first else ab
  _zero_loop(zb)
  pltpu.sync_copy(zb, acc_s.at[sl])
  plsc.subcore_barrier()

  _edge_phase(cid, sid, roww, colw, y_s, acc_s, rbufs, cbufs, vbufs,
              sems[:NBUF], sems[NBUF:2 * NBUF], sems[2 * NBUF])

  plsc.subcore_barrier()
  pltpu.sync_copy(acc_s.at[sl], zpart.at[cid, sl])


def _final_body(pa, yp, dv, psum, ab, bb, cb, db, ps):
  cid = lax.axis_index("c")
  sid = lax.axis_index("s")
  sl = pl.ds(sid * NSL, NSL)
  pltpu.sync_copy(pa.at[0, sl], ab)
  pltpu.sync_copy(pa.at[1, sl], bb)
  pltpu.sync_copy(yp.at[sl], cb)
  pltpu.sync_copy(dv.at[sl], db)
  ps[...] = jnp.zeros((LANES,), _f32)

  def ew_body(j, _):
    v = pl.ds(j * LANES, LANES)
    x = db[v] * (ab[v] + bb[v] + cb[v])
    ps[...] = ps[...] + x * x
    return 0

  lax.fori_loop(0, VLOOP, ew_body, 0)

  @pl.when(cid == 0)
  def _():
    pltpu.sync_copy(ps, psum.at[sid])


_node_f32 = jax.ShapeDtypeStruct((NPAD,), _f32)
_part_f32 = jax.ShapeDtypeStruct((NC, NPAD), _f32)
_psum_t = jax.ShapeDtypeStruct((NS, LANES), _f32)

_edge_scratch = [
    pltpu.MemorySpace.VMEM_SHARED((NPAD,), _f32),
    pltpu.MemorySpace.VMEM_SHARED((NPAD,), _f32),
    pltpu.VMEM((NSL,), _f32),
    pltpu.VMEM((NSL,), _f32),
    pltpu.VMEM((NSL,), _f32),
    pltpu.VMEM((NSL,), _f32),
    pltpu.VMEM((NSL,), _f32),
] + [pltpu.VMEM((EBLK,), _i32)] * NBUF \
  + [pltpu.VMEM((EBLK,), _i32)] * NBUF \
  + [pltpu.VMEM((EBLK,), _f32)] * NBUF + [
    pltpu.VMEM((LANES,), _f32),
] + [pltpu.SemaphoreType.DMA] * (2 * NBUF + 1)

_hist_kernel = pl.kernel(
    _hist_body,
    out_type=_part_f32,
    mesh=_mesh,
    scratch_types=[
        pltpu.MemorySpace.VMEM_SHARED((NPAD,), _f32),
        pltpu.VMEM((NSL,), _f32),
    ] + [pltpu.VMEM((EBLK,), _i32)] * NBUF + [
        pltpu.VMEM((EBLK,), _f32),
    ] + [pltpu.SemaphoreType.DMA] * (2 * NBUF),
)

_layer1_kernel = pl.kernel(
    functools.partial(_layer_body, True),
    out_type=_part_f32,
    mesh=_mesh,
    scratch_types=_edge_scratch,
)


def _dinv_tc_body(dp_ref, hp_ref, dinv_ref, y0_ref):
  d = dp_ref[0] + dp_ref[1] + 1.0
  dv = jax.lax.rsqrt(d)
  dinv_ref[...] = dv
  y0_ref[...] = dv * hp_ref[...]


_R2 = NPAD // 128
_dinv_tc = pl.pallas_call(
    _dinv_tc_body,
    out_shape=(
        jax.ShapeDtypeStruct((_R2, 128), _f32),
        jax.ShapeDtypeStruct((_R2, 128), _f32),
    ),
)

_layer_kernel = pl.kernel(
    functools.partial(_layer_body, False),
    out_type=(_node_f32, _psum_t, _part_f32),
    mesh=_mesh,
    scratch_types=_edge_scratch,
)

_final_kernel = pl.kernel(
    _final_body,
    out_type=_psum_t,
    mesh=_mesh,
    scratch_types=[
        pltpu.VMEM((NSL,), _f32),
        pltpu.VMEM((NSL,), _f32),
        pltpu.VMEM((NSL,), _f32),
        pltpu.VMEM((NSL,), _f32),
        pltpu.VMEM((LANES,), _f32),
    ],
)


@jax.jit
def kernel(h, edge_index):
  epad = EPAD - E
  fill = jnp.arange(epad, dtype=_i32)
  row = jnp.concatenate([edge_index[0], fill % N])
  col = jnp.concatenate([edge_index[1], N + fill % (NPAD - N)])
  roww = row.reshape(EROWS, EBLK)
  colw = col.reshape(EROWS, EBLK)
  hp = jnp.concatenate([h[:, 0], jnp.zeros((NPAD - N,), _f32)])

  dpart = _hist_kernel(colw)
  dinv2, y02 = _dinv_tc(dpart.reshape(NC, _R2, 128), hp.reshape(_R2, 128))
  dinv = dinv2.reshape(NPAD)
  y0 = y02.reshape(NPAD)
  z1 = _layer1_kernel(roww, colw, y0)
  y1, ps1, z2 = _layer_kernel(roww, colw, z1, y0, dinv)
  y2, ps2, z3 = _layer_kernel(roww, colw, z2, y1, dinv)
  ps3 = _final_kernel(z3, y2, dinv)
  return jnp.stack([jnp.sum(ps1), jnp.sum(ps2), jnp.sum(ps3)])


# speedup vs baseline: 387.7780x; 1.0002x over previous
import functools

import jax
import jax.numpy as jnp
from jax import lax
from jax.experimental import pallas as pl
from jax.experimental.pallas import tpu as pltpu
from jax.experimental.pallas import tpu_sc as plsc

N = 100000
NC = 2
NS = 16
LANES = 16

NPAD = 100352
NSL = NPAD // NS
VLOOP = NSL // LANES

E = 6400000
NBUF = 4
NBLKS = 100
EBLK = 2048
EPAD = NC * NS * NBLKS * EBLK
EROWS = EPAD // EBLK

_mesh = plsc.VectorSubcoreMesh(
    core_axis_name="c", subcore_axis_name="s", num_cores=NC, num_subcores=NS)

_f32 = jnp.float32
_i32 = jnp.int32


def _zero_loop(ref):
  def body(j, _):
    ref[pl.ds(j * LANES, LANES)] = jnp.zeros((LANES,), _f32)
    return 0
  lax.fori_loop(0, VLOOP, body, 0)


def _edge_phase(cid, sid, roww, colw, y_s, acc_s, rbufs, cbufs, vbufs,
                semL, semS, semG, ones=None):
  hist = ones is not None
  wrbase = (cid * NS + sid) * NBLKS

  def load(b, p):
    wr = wrbase + b
    if not hist:
      pltpu.async_copy(roww.at[wr], rbufs[p], semL[p])
    pltpu.async_copy(colw.at[wr], cbufs[p], semL[p])

  def wait_load(b, p):
    wr = wrbase + b
    if not hist:
      pltpu.make_async_copy(roww.at[wr], rbufs[p], semL[p]).wait()
    pltpu.make_async_copy(colw.at[wr], cbufs[p], semL[p]).wait()

  def gathers(p):
    pltpu.async_copy(y_s.at[rbufs[p]], vbufs[p], semG)
    pltpu.make_async_copy(y_s.at[rbufs[p]], vbufs[p], semG).wait()

  def scatters_fire(p):
    src = ones if hist else vbufs[p]
    pltpu.async_copy(src, acc_s.at[cbufs[p]], semS[p], add=True)

  def scatters_drain(p):
    src = ones if hist else vbufs[p]
    pltpu.make_async_copy(src, acc_s.at[cbufs[p]], semS[p]).wait()

  for b in range(NBUF):
    load(b, b)

  def g_body(g, _):
    for par in range(NBUF):
      c = NBUF * g + par
      p = (par + 1) % NBUF

      @pl.when(c >= NBUF - 1)
      def _():
        scatters_drain(p)

      nxt = c + 1

      @pl.when(jnp.logical_and(nxt >= NBUF, nxt < NBLKS))
      def _():
        load(nxt, p)

      wait_load(c, par)
      if not hist:
        gathers(par)
      scatters_fire(par)
    return 0

  lax.fori_loop(0, NBLKS // NBUF, g_body, 0)
  for par in range(1, NBUF):
    scatters_drain(par)


def _hist_body(colw, dpart, deg_s, zbuf, *rest):
  cbufs = rest[:NBUF]
  ones = rest[NBUF]
  sems = rest[NBUF + 1:]
  cid = lax.axis_index("c")
  sid = lax.axis_index("s")
  base = sid * NSL

  _zero_loop(zbuf)

  def ones_body(w, _):
    ones[pl.ds(w * LANES, LANES)] = jnp.ones((LANES,), _f32)
    return 0

  lax.fori_loop(0, EBLK // LANES, ones_body, 0)
  pltpu.sync_copy(zbuf, deg_s.at[pl.ds(base, NSL)])
  plsc.subcore_barrier()

  semL = sems[:NBUF]
  semS = sems[NBUF:2 * NBUF]
  _edge_phase(cid, sid, None, colw, None, deg_s, None, cbufs, None,
              semL, semS, None, ones=ones)

  plsc.subcore_barrier()
  pltpu.sync_copy(deg_s.at[pl.ds(base, NSL)], dpart.at[cid, pl.ds(base, NSL)])


def _layer_body(first, *refs):
  if first:
    (roww, colw, y0, zpart,
     y_s, acc_s, ab, bb, cb, db, ob, *rest) = refs
  else:
    (roww, colw, pa, yp, dv, y_o, psum, zpart,
     y_s, acc_s, ab, bb, cb, db, ob, *rest) = refs
  rbufs = rest[:NBUF]
  cbufs = rest[NBUF:2 * NBUF]
  vbufs = rest[2 * NBUF:3 * NBUF]
  ps = rest[3 * NBUF]
  sems = rest[3 * NBUF + 1:]
  cid = lax.axis_index("c")
  sid = lax.axis_index("s")
  base = sid * NSL
  sl = pl.ds(base, NSL)

  if first:
    pltpu.sync_copy(y0.at[sl], y_s.at[sl])
  else:
    pltpu.sync_copy(pa.at[0, sl], ab)
    pltpu.sync_copy(pa.at[1, sl], bb)
    pltpu.sync_copy(yp.at[sl], cb)
    pltpu.sync_copy(dv.at[sl], db)
    ps[...] = jnp.zeros((LANES,), _f32)

    def ew_body(j, _):
      v = pl.ds(j * LANES, LANES)
      dvv = db[v]
      x = dvv * (ab[v] + bb[v] + cb[v])
      ps[...] = ps[...] + x * x
      ob[v] = dvv * x
      return 0

    lax.fori_loop(0, VLOOP, ew_body, 0)

    pltpu.sync_copy(ob, y_s.at[sl])

    @pl.when(cid == 0)
    def _():
      pltpu.sync_copy(ob, y_o.at[sl])
      pltpu.sync_copy(ps, psum.at[sid])

  zb = bb if first else ab
  _zero_loop(zb)
  pltpu.sync_copy(zb, acc_s.at[sl])
  plsc.subcore_barrier()

  _edge_phase(cid, sid, roww, colw, y_s, acc_s, rbufs, cbufs, vbufs,
              sems[:NBUF], sems[NBUF:2 * NBUF], sems[2 * NBUF])

  plsc.subcore_barrier()
  pltpu.sync_copy(acc_s.at[sl], zpart.at[cid, sl])


def _final_body(pa, yp, dv, psum, ab, bb, cb, db, ps):
  cid = lax.axis_index("c")
  sid = lax.axis_index("s")
  sl = pl.ds(sid * NSL, NSL)
  pltpu.sync_copy(pa.at[0, sl], ab)
  pltpu.sync_copy(pa.at[1, sl], bb)
  pltpu.sync_copy(yp.at[sl], cb)
  pltpu.sync_copy(dv.at[sl], db)
  ps[...] = jnp.zeros((LANES,), _f32)

  def ew_body(j, _):
    v = pl.ds(j * LANES, LANES)
    x = db[v] * (ab[v] + bb[v] + cb[v])
    ps[...] = ps[...] + x * x
    return 0

  lax.fori_loop(0, VLOOP, ew_body, 0)

  @pl.when(cid == 0)
  def _():
    pltpu.sync_copy(ps, psum.at[sid])


_node_f32 = jax.ShapeDtypeStruct((NPAD,), _f32)
_part_f32 = jax.ShapeDtypeStruct((NC, NPAD), _f32)
_psum_t = jax.ShapeDtypeStruct((NS, LANES), _f32)

_edge_scratch = [
    pltpu.MemorySpace.VMEM_SHARED((NPAD,), _f32),
    pltpu.MemorySpace.VMEM_SHARED((NPAD,), _f32),
    pltpu.VMEM((NSL,), _f32),
    pltpu.VMEM((NSL,), _f32),
    pltpu.VMEM((NSL,), _f32),
    pltpu.VMEM((NSL,), _f32),
    pltpu.VMEM((NSL,), _f32),
] + [pltpu.VMEM((EBLK,), _i32)] * NBUF \
  + [pltpu.VMEM((EBLK,), _i32)] * NBUF \
  + [pltpu.VMEM((EBLK,), _f32)] * NBUF + [
    pltpu.VMEM((LANES,), _f32),
] + [pltpu.SemaphoreType.DMA] * (2 * NBUF + 1)

_hist_kernel = pl.kernel(
    _hist_body,
    out_type=_part_f32,
    mesh=_mesh,
    scratch_types=[
        pltpu.MemorySpace.VMEM_SHARED((NPAD,), _f32),
        pltpu.VMEM((NSL,), _f32),
    ] + [pltpu.VMEM((EBLK,), _i32)] * NBUF + [
        pltpu.VMEM((EBLK,), _f32),
    ] + [pltpu.SemaphoreType.DMA] * (2 * NBUF),
)

_layer1_kernel = pl.kernel(
    functools.partial(_layer_body, True),
    out_type=_part_f32,
    mesh=_mesh,
    scratch_types=_edge_scratch,
)

_layer_kernel = pl.kernel(
    functools.partial(_layer_body, False),
    out_type=(_node_f32, _psum_t, _part_f32),
    mesh=_mesh,
    scratch_types=_edge_scratch,
)

_final_kernel = pl.kernel(
    _final_body,
    out_type=_psum_t,
    mesh=_mesh,
    scratch_types=[
        pltpu.VMEM((NSL,), _f32),
        pltpu.VMEM((NSL,), _f32),
        pltpu.VMEM((NSL,), _f32),
        pltpu.VMEM((NSL,), _f32),
        pltpu.VMEM((LANES,), _f32),
    ],
)


def _dinv_tc_body(dp_ref, hp_ref, dinv_ref, y0_ref):
  d = dp_ref[0] + dp_ref[1] + 1.0
  dv = jax.lax.rsqrt(d)
  dinv_ref[...] = dv
  y0_ref[...] = dv * hp_ref[...]


_R2 = NPAD // 128
_dinv_tc = pl.pallas_call(
    _dinv_tc_body,
    out_shape=(
        jax.ShapeDtypeStruct((_R2, 128), _f32),
        jax.ShapeDtypeStruct((_R2, 128), _f32),
    ),
)


@jax.jit
def kernel(h, edge_index):
  epad = EPAD - E
  fill = jnp.arange(epad, dtype=_i32)
  row = jnp.concatenate([edge_index[0], fill % N])
  col = jnp.concatenate([edge_index[1], N + fill % (NPAD - N)])
  roww = row.reshape(EROWS, EBLK)
  colw = col.reshape(EROWS, EBLK)
  hp = jnp.concatenate([h[:, 0], jnp.zeros((NPAD - N,), _f32)])

  dpart = _hist_kernel(colw)
  dinv2, y02 = _dinv_tc(dpart.reshape(NC, _R2, 128), hp.reshape(_R2, 128))
  dinv = dinv2.reshape(NPAD)
  y0 = y02.reshape(NPAD)
  z1 = _layer1_kernel(roww, colw, y0)
  y1, ps1, z2 = _layer_kernel(roww, colw, z1, y0, dinv)
  y2, ps2, z3 = _layer_kernel(roww, colw, z2, y1, dinv)
  ps3 = _final_kernel(z3, y2, dinv)
  return jnp.stack([jnp.sum(ps1), jnp.sum(ps2), jnp.sum(ps3)])


# speedup vs baseline: 433.0649x; 1.1168x over previous
import functools

import jax
import jax.numpy as jnp
from jax import lax
from jax.experimental import pallas as pl
from jax.experimental.pallas import tpu as pltpu
from jax.experimental.pallas import tpu_sc as plsc

N = 100000
NC = 2
NS = 16
LANES = 16

NPAD = 100352
NTAB = 100000
NSL = NPAD // NS
VLOOP = NSL // LANES
QCH = 7
QLEN = NSL // QCH
QVL = QLEN // LANES

E = 6400000
NBUF = 4
NBLKS = 200
EBLK = 1024
EPAD = NC * NS * NBLKS * EBLK
EROWS = EPAD // EBLK

_mesh = plsc.VectorSubcoreMesh(
    core_axis_name="c", subcore_axis_name="s", num_cores=NC, num_subcores=NS)

_f32 = jnp.float32
_i32 = jnp.int32
_sc_params = pltpu.CompilerParams(needs_layout_passes=False)


def _zero_loop(ref, iters):
  def body(j, _):
    ref[pl.ds(j * LANES, LANES)] = jnp.zeros((LANES,), _f32)
    return 0
  lax.fori_loop(0, iters, body, 0)


def _edge_phase(cid, sid, roww, colw, y_s, acc_s, rbufs, cbufs, vbufs,
                semL, semS, semG, ones=None, y_tab=None):
  hist = ones is not None
  wrbase = (cid * NS + sid) * NBLKS

  def load(b, p):
    esl = pl.ds((wrbase + b) * EBLK, EBLK)
    if not hist:
      pltpu.async_copy(roww.at[esl], rbufs[p], semL[p])
    pltpu.async_copy(colw.at[esl], cbufs[p], semL[p])

  def wait_load(b, p):
    esl = pl.ds((wrbase + b) * EBLK, EBLK)
    if not hist:
      pltpu.make_async_copy(roww.at[esl], rbufs[p], semL[p]).wait()
    pltpu.make_async_copy(colw.at[esl], cbufs[p], semL[p]).wait()

  def gathers(p):
    if y_tab is None:
      pltpu.async_copy(y_s.at[rbufs[p]], vbufs[p], semG)
      pltpu.make_async_copy(y_s.at[rbufs[p]], vbufs[p], semG).wait()
    else:
      def body(i, _):
        v = pl.ds(i * LANES, LANES)
        vbufs[p][v] = plsc.load_gather(y_tab, [rbufs[p][v]])
        return 0
      lax.fori_loop(0, EBLK // LANES, body, 0)

  def scatters_fire(p):
    src = ones if hist else vbufs[p]
    pltpu.async_copy(src, acc_s.at[cbufs[p]], semS[p], add=True)

  def scatters_drain(p):
    src = ones if hist else vbufs[p]
    pltpu.make_async_copy(src, acc_s.at[cbufs[p]], semS[p]).wait()

  for b in range(NBUF):
    load(b, b)

  def g_body(g, _):
    for par in range(NBUF):
      c = NBUF * g + par
      p = (par + 1) % NBUF

      @pl.when(c >= NBUF - 1)
      def _():
        scatters_drain(p)

      nxt = c + 1

      @pl.when(jnp.logical_and(nxt >= NBUF, nxt < NBLKS))
      def _():
        load(nxt, p)

      wait_load(c, par)
      if not hist:
        gathers(par)
      scatters_fire(par)
    return 0

  lax.fori_loop(0, NBLKS // NBUF, g_body, 0)
  for par in range(1, NBUF):
    scatters_drain(par)


def _hist_body(colw, dpart, deg_s, zbuf, *rest):
  cbufs = rest[:NBUF]
  ones = rest[NBUF]
  sems = rest[NBUF + 1:]
  cid = lax.axis_index("c")
  sid = lax.axis_index("s")
  base = sid * NSL

  _zero_loop(zbuf, VLOOP)

  def ones_body(w, _):
    ones[pl.ds(w * LANES, LANES)] = jnp.ones((LANES,), _f32)
    return 0

  lax.fori_loop(0, EBLK // LANES, ones_body, 0)
  pltpu.sync_copy(zbuf, deg_s.at[pl.ds(base, NSL)])
  plsc.subcore_barrier()

  semL = sems[:NBUF]
  semS = sems[NBUF:2 * NBUF]
  _edge_phase(cid, sid, None, colw, None, deg_s, None, cbufs, None,
              semL, semS, None, ones=ones)

  plsc.subcore_barrier()
  pltpu.sync_copy(deg_s.at[pl.ds(base, NSL)],
                  dpart.at[pl.ds(cid * NPAD + base, NSL)])


def _layer_body(first, *refs):
  if first:
    (roww, colw, y0, zpart,
     y_s, acc_s, y_t, ab, bb, cb, db, *rest) = refs
  else:
    (roww, colw, pa, yp, dv, y_o, psum, zpart,
     y_s, acc_s, y_t, ab, bb, cb, db, *rest) = refs
  rbufs = rest[:NBUF]
  cbufs = rest[NBUF:2 * NBUF]
  vbufs = rest[2 * NBUF:3 * NBUF]
  ps = rest[3 * NBUF]
  sems = rest[3 * NBUF + 1:]
  cid = lax.axis_index("c")
  sid = lax.axis_index("s")
  base = sid * NSL
  sl = pl.ds(base, NSL)

  if first:
    pltpu.sync_copy(y0.at[sl], y_s.at[sl])
  else:
    ps[...] = jnp.zeros((LANES,), _f32)

    def chunk_body(q, _):
      qsl = pl.ds(base + q * QLEN, QLEN)
      pltpu.sync_copy(pa.at[pl.ds(base + q * QLEN, QLEN)], ab)
      pltpu.sync_copy(pa.at[pl.ds(NPAD + base + q * QLEN, QLEN)], bb)
      pltpu.sync_copy(yp.at[qsl], cb)
      pltpu.sync_copy(dv.at[qsl], db)

      def ew_body(j, _):
        v = pl.ds(j * LANES, LANES)
        dvv = db[v]
        x = dvv * (ab[v] + bb[v] + cb[v])
        ps[...] = ps[...] + x * x
        db[v] = dvv * x
        return 0

      lax.fori_loop(0, QVL, ew_body, 0)

      pltpu.sync_copy(db, y_s.at[qsl])

      @pl.when(cid == 0)
      def _():
        pltpu.sync_copy(db, y_o.at[qsl])
      return 0

    lax.fori_loop(0, QCH, chunk_body, 0)

    @pl.when(cid == 0)
    def _():
      pltpu.sync_copy(ps, psum.at[sid])

  zb = bb if first else ab
  _zero_loop(zb, QVL)

  def zacc_body(q, _):
    pltpu.sync_copy(zb, acc_s.at[pl.ds(base + q * QLEN, QLEN)])
    return 0

  lax.fori_loop(0, QCH, zacc_body, 0)
  plsc.subcore_barrier()

  def ytab_body(t, _):
    tsl = pl.ds(t * 4000, 4000)
    pltpu.sync_copy(y_s.at[tsl], y_t.at[tsl])
    return 0

  lax.fori_loop(0, NTAB // 4000, ytab_body, 0)

  _edge_phase(cid, sid, roww, colw, y_s, acc_s, rbufs, cbufs, vbufs,
              sems[:NBUF], sems[NBUF:2 * NBUF], sems[2 * NBUF],
              y_tab=y_t)

  plsc.subcore_barrier()
  pltpu.sync_copy(acc_s.at[sl], zpart.at[pl.ds(cid * NPAD + base, NSL)])


def _final_body(pa, yp, dv, psum, ab, bb, cb, db, ps):
  cid = lax.axis_index("c")
  sid = lax.axis_index("s")
  sl = pl.ds(sid * NSL, NSL)
  pltpu.sync_copy(pa.at[sl], ab)
  pltpu.sync_copy(pa.at[pl.ds(NPAD + sid * NSL, NSL)], bb)
  pltpu.sync_copy(yp.at[sl], cb)
  pltpu.sync_copy(dv.at[sl], db)
  ps[...] = jnp.zeros((LANES,), _f32)

  def ew_body(j, _):
    v = pl.ds(j * LANES, LANES)
    x = db[v] * (ab[v] + bb[v] + cb[v])
    ps[...] = ps[...] + x * x
    return 0

  lax.fori_loop(0, VLOOP, ew_body, 0)

  @pl.when(cid == 0)
  def _():
    pltpu.sync_copy(ps, psum.at[sid])


_node_f32 = jax.ShapeDtypeStruct((NPAD,), _f32)
_part_f32 = jax.ShapeDtypeStruct((NC * NPAD,), _f32)
_psum_t = jax.ShapeDtypeStruct((NS, LANES), _f32)

_edge_scratch = [
    pltpu.MemorySpace.VMEM_SHARED((NPAD,), _f32),
    pltpu.MemorySpace.VMEM_SHARED((NPAD,), _f32),
    pltpu.VMEM((NTAB,), _f32),
    pltpu.VMEM((QLEN,), _f32),
    pltpu.VMEM((QLEN,), _f32),
    pltpu.VMEM((QLEN,), _f32),
    pltpu.VMEM((QLEN,), _f32),
] + [pltpu.VMEM((EBLK,), _i32)] * NBUF \
  + [pltpu.VMEM((EBLK,), _i32)] * NBUF \
  + [pltpu.VMEM((EBLK,), _f32)] * NBUF + [
    pltpu.VMEM((LANES,), _f32),
] + [pltpu.SemaphoreType.DMA] * (2 * NBUF + 1)

_hist_kernel = pl.kernel(
    _hist_body,
    out_type=_part_f32,
    mesh=_mesh,
    scratch_types=[
        pltpu.MemorySpace.VMEM_SHARED((NPAD,), _f32),
        pltpu.VMEM((NSL,), _f32),
    ] + [pltpu.VMEM((EBLK,), _i32)] * NBUF + [
        pltpu.VMEM((EBLK,), _f32),
    ] + [pltpu.SemaphoreType.DMA] * (2 * NBUF),
)

_layer1_kernel = pl.kernel(
    functools.partial(_layer_body, True),
    out_type=_part_f32,
    mesh=_mesh,
    compiler_params=_sc_params,
    scratch_types=_edge_scratch,
)

_layer_kernel = pl.kernel(
    functools.partial(_layer_body, False),
    out_type=(_node_f32, _psum_t, _part_f32),
    mesh=_mesh,
    compiler_params=_sc_params,
    scratch_types=_edge_scratch,
)

_final_kernel = pl.kernel(
    _final_body,
    out_type=_psum_t,
    mesh=_mesh,
    scratch_types=[
        pltpu.VMEM((NSL,), _f32),
        pltpu.VMEM((NSL,), _f32),
        pltpu.VMEM((NSL,), _f32),
        pltpu.VMEM((NSL,), _f32),
        pltpu.VMEM((LANES,), _f32),
    ],
)


def _dinv_tc_body(dp_ref, hp_ref, dinv_ref, y0_ref):
  d = dp_ref[0] + dp_ref[1] + 1.0
  dv = jax.lax.rsqrt(d)
  dinv_ref[...] = dv
  y0_ref[...] = dv * hp_ref[...]


_R2 = NPAD // 128
_dinv_tc = pl.pallas_call(
    _dinv_tc_body,
    out_shape=(
        jax.ShapeDtypeStruct((_R2, 128), _f32),
        jax.ShapeDtypeStruct((_R2, 128), _f32),
    ),
)


@jax.jit
def kernel(h, edge_index):
  epad = EPAD - E
  fill = jnp.arange(epad, dtype=_i32)
  row = jnp.concatenate([edge_index[0], fill % N])
  col = jnp.concatenate([edge_index[1], N + fill % (NPAD - N)])
  hp = jnp.concatenate([h[:, 0], jnp.zeros((NPAD - N,), _f32)])

  dpart = _hist_kernel(col)
  dinv2, y02 = _dinv_tc(dpart.reshape(NC, _R2, 128), hp.reshape(_R2, 128))
  dinv = dinv2.reshape(NPAD)
  y0 = y02.reshape(NPAD)
  z1 = _layer1_kernel(row, col, y0)
  y1, ps1, z2 = _layer_kernel(row, col, z1, y0, dinv)
  y2, ps2, z3 = _layer_kernel(row, col, z2, y1, dinv)
  ps3 = _final_kernel(z3, y2, dinv)
  return jnp.stack([jnp.sum(ps1), jnp.sum(ps2), jnp.sum(ps3)])
